# Initial kernel scaffold; baseline (speedup 1.0000x reference)
import jax
import jax.numpy as jnp
from jax.experimental import pallas as pl


def kernel(cons_embedding, vals_embedding, cons_embedding_0, vals_embedding_0, v2c_edge_index, c2v_edge_index, v2c_edge_attr, c2v_edge_attr, cons_batch, vals_batch, edge_norm, Wm_v2c, bm_v2c, Wu_v2c, bu_v2c, Wm_c2v, bm_c2v, Wu_c2v, bu_c2v):
    raise NotImplementedError("write your pallas kernel here")


# speedup vs baseline: 2.4492x; 2.4492x over previous
import functools

import jax
import jax.numpy as jnp
from jax import lax
from jax.experimental import pallas as pl
from jax.experimental.pallas import tpu as pltpu
from jax.experimental.pallas import tpu_sc as plsc

D = 128
DE = 16
N = 10000
E = 320000
NPAD = 10240

NC = 2
NS = 16
C = 80


def _linear_body(x_ref, w_ref, b_ref, o_ref):
    o_ref[...] = (
        jnp.dot(x_ref[...], w_ref[...], preferred_element_type=jnp.float32)
        + b_ref[...]
    )


def _tc_linear(x, w, b, block_rows):
    n = x.shape[0]
    k = x.shape[1]
    return pl.pallas_call(
        _linear_body,
        grid=(n // block_rows,),
        in_specs=[
            pl.BlockSpec((block_rows, k), lambda i: (i, 0)),
            pl.BlockSpec((k, D), lambda i: (0, 0)),
            pl.BlockSpec((1, D), lambda i: (0, 0)),
        ],
        out_specs=pl.BlockSpec((block_rows, D), lambda i: (i, 0)),
        out_shape=jax.ShapeDtypeStruct((n, D), jnp.float32),
    )(x, w, b.reshape(1, D))


def _update_body(x_ref, a0_ref, a1_ref, wu_ref, bu_ref, x0_ref, o_ref):
    agg = a0_ref[...] + a1_ref[...]
    h = (
        jnp.dot(x_ref[...], wu_ref[0:D, :], preferred_element_type=jnp.float32)
        + jnp.dot(agg, wu_ref[D : 2 * D, :], preferred_element_type=jnp.float32)
        + bu_ref[...]
    )
    o_ref[...] = jnp.maximum(h, 0.0) + x0_ref[...]


def _tc_update(x_dst, agg2, wu, bu, x_0, block_rows):
    n = x_dst.shape[0]
    row_spec = pl.BlockSpec((block_rows, D), lambda i: (i, 0))
    return pl.pallas_call(
        _update_body,
        grid=(n // block_rows,),
        in_specs=[
            row_spec,
            pl.BlockSpec((block_rows, D), lambda i: (i, 0)),
            pl.BlockSpec((block_rows, D), lambda i: (i, 0)),
            pl.BlockSpec((2 * D, D), lambda i: (0, 0)),
            pl.BlockSpec((1, D), lambda i: (0, 0)),
            row_spec,
        ],
        out_specs=row_spec,
        out_shape=jax.ShapeDtypeStruct((n, D), jnp.float32),
    )(x_dst, agg2[0], agg2[1], wu, bu.reshape(1, D), x_0)


def _sc_body(p_hbm, a_hbm, src_hbm, dst_hbm, nrm_hbm, zeros_hbm, agg_hbm,
             acc_sh, src_v, dst_v, nrm_v, pg_v, a_v, sem):
    c = lax.axis_index("c")
    s = lax.axis_index("s")

    rows_per_tile = NPAD // NS
    rsl = pl.ds(s * rows_per_tile, rows_per_tile)
    pltpu.sync_copy(zeros_hbm.at[rsl, :], acc_sh.at[rsl, :])
    plsc.subcore_barrier()

    ec = E // NC
    et = ec // NS
    base0 = c * ec + s * et
    nchunks = et // C

    def chunk_body(kk, carry):
        base = base0 + kk * C
        pltpu.sync_copy(src_hbm.at[pl.ds(base, C)], src_v)
        gather = pltpu.async_copy(p_hbm.at[src_v], pg_v, sem)
        pltpu.sync_copy(dst_hbm.at[pl.ds(base, C)], dst_v)
        pltpu.sync_copy(nrm_hbm.at[pl.ds(base, C)], nrm_v)
        pltpu.sync_copy(a_hbm.at[pl.ds(base, C), :], a_v)
        gather.wait()

        def group_body(g, cc):
            nv = nrm_v[pl.ds(g * 16, 16)]
            for j in range(16):
                e = g * 16 + j
                nrm = lax.gather(
                    nv,
                    jnp.full((16, 1), j, jnp.int32),
                    lax.GatherDimensionNumbers(
                        offset_dims=(),
                        collapsed_slice_dims=(0,),
                        start_index_map=(0,),
                    ),
                    slice_sizes=(1,),
                    mode=lax.GatherScatterMode.PROMISE_IN_BOUNDS,
                )
                for f in range(D // 16):
                    sl = pl.ds(f * 16, 16)
                    m = jnp.maximum(pg_v[e, sl] + a_v[e, sl], 0.0) * nrm
                    pg_v[e, sl] = m
            return cc

        lax.fori_loop(0, C // 16, group_body, 0)
        pltpu.sync_copy(pg_v, acc_sh.at[dst_v], add=True)
        return carry

    lax.fori_loop(0, nchunks, chunk_body, 0)

    plsc.subcore_barrier()
    pltpu.sync_copy(acc_sh.at[rsl, :], agg_hbm.at[c, rsl, :])


@functools.partial(
    pl.kernel,
    out_type=jax.ShapeDtypeStruct((NC, NPAD, D), jnp.float32),
    mesh=plsc.VectorSubcoreMesh(core_axis_name="c", subcore_axis_name="s"),
    scratch_types=[
        pltpu.VMEM_SHARED((NPAD, D), jnp.float32),
        pltpu.VMEM((C,), jnp.int32),
        pltpu.VMEM((C,), jnp.int32),
        pltpu.VMEM((C,), jnp.float32),
        pltpu.VMEM((C, D), jnp.float32),
        pltpu.VMEM((C, D), jnp.float32),
        pltpu.SemaphoreType.DMA,
    ],
)
def _sc_gather_scatter(p_hbm, a_hbm, src_hbm, dst_hbm, nrm_hbm, zeros_hbm,
                       agg_hbm, acc_sh, src_v, dst_v, nrm_v, pg_v, a_v, sem):
    _sc_body(p_hbm, a_hbm, src_hbm, dst_hbm, nrm_hbm, zeros_hbm, agg_hbm,
             acc_sh, src_v, dst_v, nrm_v, pg_v, a_v, sem)


def _pass(x_src, x_dst, x_0, edge_index, edge_attr, edge_norm, wm, bm, wu, bu,
          zeros):
    p = _tc_linear(x_src, wm[:D], bm, block_rows=2000)
    a = _tc_linear(edge_attr, wm[D:], jnp.zeros((D,), jnp.float32),
                   block_rows=2000)
    src = edge_index[0].astype(jnp.int32)
    dst = edge_index[1].astype(jnp.int32)
    agg2 = _sc_gather_scatter(p, a, src, dst, edge_norm, zeros)
    return _tc_update(x_dst, agg2, wu, bu, x_0, block_rows=2000)


def kernel(cons_embedding, vals_embedding, cons_embedding_0, vals_embedding_0,
           v2c_edge_index, c2v_edge_index, v2c_edge_attr, c2v_edge_attr,
           cons_batch, vals_batch, edge_norm, Wm_v2c, bm_v2c, Wu_v2c, bu_v2c,
           Wm_c2v, bm_c2v, Wu_c2v, bu_c2v):
    del cons_batch, vals_batch
    zeros = jnp.zeros((NPAD, D), jnp.float32)
    cons_new = _pass(vals_embedding, cons_embedding, cons_embedding_0,
                     v2c_edge_index, v2c_edge_attr, edge_norm,
                     Wm_v2c, bm_v2c, Wu_v2c, bu_v2c, zeros)
    vals_new = _pass(cons_new, vals_embedding, vals_embedding_0,
                     c2v_edge_index, c2v_edge_attr, edge_norm,
                     Wm_c2v, bm_c2v, Wu_c2v, bu_c2v, zeros)
    return (vals_new, cons_new)


# speedup vs baseline: 2.7802x; 1.1351x over previous
import functools

import jax
import jax.numpy as jnp
from jax import lax
from jax.experimental import pallas as pl
from jax.experimental.pallas import tpu as pltpu
from jax.experimental.pallas import tpu_sc as plsc

D = 128
DE = 16
N = 10000
E = 320000
NPAD = 10240

NC = 2
NS = 16
C = 40


def _linear_body(x_ref, w_ref, b_ref, o_ref):
    o_ref[...] = (
        jnp.dot(x_ref[...], w_ref[...], preferred_element_type=jnp.float32)
        + b_ref[...]
    )


def _tc_linear(x, w, b, block_rows):
    n = x.shape[0]
    k = x.shape[1]
    return pl.pallas_call(
        _linear_body,
        grid=(n // block_rows,),
        in_specs=[
            pl.BlockSpec((block_rows, k), lambda i: (i, 0)),
            pl.BlockSpec((k, D), lambda i: (0, 0)),
            pl.BlockSpec((1, D), lambda i: (0, 0)),
        ],
        out_specs=pl.BlockSpec((block_rows, D), lambda i: (i, 0)),
        out_shape=jax.ShapeDtypeStruct((n, D), jnp.float32),
    )(x, w, b.reshape(1, D))


def _update_body(x_ref, a0_ref, a1_ref, wu_ref, bu_ref, x0_ref, o_ref):
    agg = a0_ref[...] + a1_ref[...]
    h = (
        jnp.dot(x_ref[...], wu_ref[0:D, :], preferred_element_type=jnp.float32)
        + jnp.dot(agg, wu_ref[D : 2 * D, :], preferred_element_type=jnp.float32)
        + bu_ref[...]
    )
    o_ref[...] = jnp.maximum(h, 0.0) + x0_ref[...]


def _tc_update(x_dst, agg2, wu, bu, x_0, block_rows):
    n = x_dst.shape[0]
    row_spec = pl.BlockSpec((block_rows, D), lambda i: (i, 0))
    return pl.pallas_call(
        _update_body,
        grid=(n // block_rows,),
        in_specs=[
            row_spec,
            pl.BlockSpec((block_rows, D), lambda i: (i, 0)),
            pl.BlockSpec((block_rows, D), lambda i: (i, 0)),
            pl.BlockSpec((2 * D, D), lambda i: (0, 0)),
            pl.BlockSpec((1, D), lambda i: (0, 0)),
            row_spec,
        ],
        out_specs=row_spec,
        out_shape=jax.ShapeDtypeStruct((n, D), jnp.float32),
    )(x_dst, agg2[0], agg2[1], wu, bu.reshape(1, D), x_0)


NBUF = 4
DRING = 2 * NBUF


def _lane_splat(nv, j):
    return lax.gather(
        nv,
        jnp.full((16, 1), j, jnp.int32),
        lax.GatherDimensionNumbers(
            offset_dims=(),
            collapsed_slice_dims=(0,),
            start_index_map=(0,),
        ),
        slice_sizes=(1,),
        mode=lax.GatherScatterMode.PROMISE_IN_BOUNDS,
    )


def _sc_body(p_hbm, a_hbm, src_hbm, dst_hbm, nrm_hbm, zeros_hbm, agg_hbm,
             acc_sh, src_v, dst_v, nrm_v, pg_v, a_v,
             isem, gsem, asem, ssem):
    c = lax.axis_index("c")
    s = lax.axis_index("s")

    rows_per_tile = NPAD // NS
    rsl = pl.ds(s * rows_per_tile, rows_per_tile)
    pltpu.sync_copy(zeros_hbm.at[rsl, :], acc_sh.at[rsl, :])
    plsc.subcore_barrier()

    ec = E // NC
    et = ec // NS
    base0 = c * ec + s * et
    nchunks = et // C

    def idx_copies(i, bi, bd):
        base = base0 + i * C
        return (
            pltpu.make_async_copy(src_hbm.at[pl.ds(base, C)], src_v.at[bi],
                                  isem.at[bi]),
            pltpu.make_async_copy(dst_hbm.at[pl.ds(base, C)], dst_v.at[bd],
                                  isem.at[bi]),
            pltpu.make_async_copy(nrm_hbm.at[pl.ds(base, C)], nrm_v.at[bi],
                                  isem.at[bi]),
        )

    def in_copies(i, bi):
        return (
            pltpu.make_async_copy(p_hbm.at[src_v.at[bi]], pg_v.at[bi],
                                  gsem.at[bi]),
            pltpu.make_async_copy(a_hbm.at[pl.ds(base0 + i * C, C), :],
                                  a_v.at[bi], asem.at[bi]),
        )

    def scatter_copy(bi, bd):
        return pltpu.make_async_copy(pg_v.at[bi], acc_sh.at[dst_v.at[bd]],
                                     ssem.at[bi])

    for cp in idx_copies(0, 0, 0) + idx_copies(1, 1, 1):
        cp.start()
    for cp in idx_copies(0, 0, 0):
        cp.wait()
    for cp in in_copies(0, 0):
        cp.start()

    def outer_body(k2, carry):
        for b0 in range(DRING):
            i = k2 * DRING + b0
            b = b0 % NBUF

            j2 = i + 2
            bi2 = (b0 + 2) % NBUF
            bd2 = (b0 + 2) % DRING

            @pl.when(j2 < nchunks)
            def _():
                for cp in idx_copies(j2, bi2, bd2):
                    cp.start()

            j1 = i + 1
            bi1 = (b0 + 1) % NBUF
            bd1 = (b0 + 1) % DRING
            bd1_old = (bd1 + DRING - NBUF) % DRING

            @pl.when(jnp.logical_and(j1 < nchunks, j1 >= NBUF))
            def _():
                scatter_copy(bi1, bd1_old).wait()

            @pl.when(j1 < nchunks)
            def _():
                for cp in idx_copies(j1, bi1, bd1):
                    cp.wait()
                for cp in in_copies(j1, bi1):
                    cp.start()

            @pl.when(i < nchunks)
            def _():
                for cp in in_copies(i, b):
                    cp.wait()

                def edge_body(e, cc):
                    nv = nrm_v[b, pl.ds((e >> 4) << 4, 16)]
                    nrm = _lane_splat(nv, e & 15)
                    for f in range(D // 16):
                        sl = pl.ds(f * 16, 16)
                        m = (
                            jnp.maximum(pg_v[b, e, sl] + a_v[b, e, sl], 0.0)
                            * nrm
                        )
                        pg_v[b, e, sl] = m
                    return cc

                lax.fori_loop(0, C, edge_body, 0)
                pltpu.async_copy(pg_v.at[b], acc_sh.at[dst_v.at[b0]],
                                 ssem.at[b], add=True)

        return carry

    nouter = (nchunks + DRING - 1) // DRING
    lax.fori_loop(0, nouter, outer_body, 0)

    for i in range(nchunks - NBUF, nchunks):
        scatter_copy(i % NBUF, i % DRING).wait()

    plsc.subcore_barrier()
    pltpu.sync_copy(acc_sh.at[rsl, :], agg_hbm.at[c, rsl, :])


@functools.partial(
    pl.kernel,
    out_type=jax.ShapeDtypeStruct((NC, NPAD, D), jnp.float32),
    mesh=plsc.VectorSubcoreMesh(core_axis_name="c", subcore_axis_name="s"),
    scratch_types=[
        pltpu.VMEM_SHARED((NPAD, D), jnp.float32),
        pltpu.VMEM((NBUF, C), jnp.int32),
        pltpu.VMEM((DRING, C), jnp.int32),
        pltpu.VMEM((NBUF, C), jnp.float32),
        pltpu.VMEM((NBUF, C, D), jnp.float32),
        pltpu.VMEM((NBUF, C, D), jnp.float32),
        pltpu.SemaphoreType.DMA((NBUF,)),
        pltpu.SemaphoreType.DMA((NBUF,)),
        pltpu.SemaphoreType.DMA((NBUF,)),
        pltpu.SemaphoreType.DMA((NBUF,)),
    ],
)
def _sc_gather_scatter(p_hbm, a_hbm, src_hbm, dst_hbm, nrm_hbm, zeros_hbm,
                       agg_hbm, acc_sh, src_v, dst_v, nrm_v, pg_v, a_v,
                       isem, gsem, asem, ssem):
    _sc_body(p_hbm, a_hbm, src_hbm, dst_hbm, nrm_hbm, zeros_hbm, agg_hbm,
             acc_sh, src_v, dst_v, nrm_v, pg_v, a_v,
             isem, gsem, asem, ssem)


def _pass(x_src, x_dst, x_0, edge_index, edge_attr, edge_norm, wm, bm, wu, bu,
          zeros):
    p = _tc_linear(x_src, wm[:D], bm, block_rows=2000)
    a = _tc_linear(edge_attr, wm[D:], jnp.zeros((D,), jnp.float32),
                   block_rows=2000)
    src = edge_index[0].astype(jnp.int32)
    dst = edge_index[1].astype(jnp.int32)
    agg2 = _sc_gather_scatter(p, a, src, dst, edge_norm, zeros)
    return _tc_update(x_dst, agg2, wu, bu, x_0, block_rows=2000)


def kernel(cons_embedding, vals_embedding, cons_embedding_0, vals_embedding_0,
           v2c_edge_index, c2v_edge_index, v2c_edge_attr, c2v_edge_attr,
           cons_batch, vals_batch, edge_norm, Wm_v2c, bm_v2c, Wu_v2c, bu_v2c,
           Wm_c2v, bm_c2v, Wu_c2v, bu_c2v):
    del cons_batch, vals_batch
    zeros = jnp.zeros((NPAD, D), jnp.float32)
    cons_new = _pass(vals_embedding, cons_embedding, cons_embedding_0,
                     v2c_edge_index, v2c_edge_attr, edge_norm,
                     Wm_v2c, bm_v2c, Wu_v2c, bu_v2c, zeros)
    vals_new = _pass(cons_new, vals_embedding, vals_embedding_0,
                     c2v_edge_index, c2v_edge_attr, edge_norm,
                     Wm_c2v, bm_c2v, Wu_c2v, bu_c2v, zeros)
    return (vals_new, cons_new)


# speedup vs baseline: 4.4393x; 1.5968x over previous
import functools

import jax
import jax.numpy as jnp
from jax import lax
from jax.experimental import pallas as pl
from jax.experimental.pallas import tpu as pltpu
from jax.experimental.pallas import tpu_sc as plsc

D = 128
DE = 16
N = 10000
E = 320000
NPAD = 10240

NC = 2
NS = 16
C = 40


def _linear_body(x_ref, w_ref, b_ref, o_ref):
    o_ref[...] = (
        jnp.dot(x_ref[...], w_ref[...], preferred_element_type=jnp.float32)
        + b_ref[...]
    )


def _tc_linear(x, w, b, block_rows):
    n = x.shape[0]
    k = x.shape[1]
    return pl.pallas_call(
        _linear_body,
        grid=(n // block_rows,),
        in_specs=[
            pl.BlockSpec((block_rows, k), lambda i: (i, 0)),
            pl.BlockSpec((k, D), lambda i: (0, 0)),
            pl.BlockSpec((1, D), lambda i: (0, 0)),
        ],
        out_specs=pl.BlockSpec((block_rows, D), lambda i: (i, 0)),
        out_shape=jax.ShapeDtypeStruct((n, D), jnp.float32),
    )(x, w, b.reshape(1, D))


def _update_body(x_ref, a0_ref, a1_ref, wu_ref, bu_ref, x0_ref, o_ref):
    agg = a0_ref[...] + a1_ref[...]
    h = (
        jnp.dot(x_ref[...], wu_ref[0:D, :], preferred_element_type=jnp.float32)
        + jnp.dot(agg, wu_ref[D : 2 * D, :], preferred_element_type=jnp.float32)
        + bu_ref[...]
    )
    o_ref[...] = jnp.maximum(h, 0.0) + x0_ref[...]


def _tc_update(x_dst, agg2, wu, bu, x_0, block_rows):
    n = x_dst.shape[0]
    row_spec = pl.BlockSpec((block_rows, D), lambda i: (i, 0))
    return pl.pallas_call(
        _update_body,
        grid=(n // block_rows,),
        in_specs=[
            row_spec,
            pl.BlockSpec((block_rows, D), lambda i: (i, 0)),
            pl.BlockSpec((block_rows, D), lambda i: (i, 0)),
            pl.BlockSpec((2 * D, D), lambda i: (0, 0)),
            pl.BlockSpec((1, D), lambda i: (0, 0)),
            row_spec,
        ],
        out_specs=row_spec,
        out_shape=jax.ShapeDtypeStruct((n, D), jnp.float32),
    )(x_dst, agg2[0], agg2[1], wu, bu.reshape(1, D), x_0)


NBUF = 4
DRING = 2 * NBUF


def _lane_splat(nv, j):
    return lax.gather(
        nv,
        jnp.full((16, 1), j, jnp.int32),
        lax.GatherDimensionNumbers(
            offset_dims=(),
            collapsed_slice_dims=(0,),
            start_index_map=(0,),
        ),
        slice_sizes=(1,),
        mode=lax.GatherScatterMode.PROMISE_IN_BOUNDS,
    )


def _sc_body(p_hbm, a_hbm, src_hbm, dst_hbm, nrm_hbm, zeros_hbm, agg_hbm,
             acc_sh, src_v, dst_v, nrm_v, pg_v, a_v,
             isem, gsem, asem, ssem):
    c = lax.axis_index("c")
    s = lax.axis_index("s")

    rows_per_tile = NPAD // NS
    rsl = pl.ds(s * rows_per_tile, rows_per_tile)
    pltpu.sync_copy(zeros_hbm.at[rsl, :], acc_sh.at[rsl, :])
    plsc.subcore_barrier()

    ec = E // NC
    et = ec // NS
    base0 = c * ec + s * et
    nchunks = et // C

    def idx_copies(i, bi, bd):
        base = base0 + i * C
        return (
            pltpu.make_async_copy(src_hbm.at[pl.ds(base, C)], src_v.at[bi],
                                  isem.at[bi]),
            pltpu.make_async_copy(dst_hbm.at[pl.ds(base, C)], dst_v.at[bd],
                                  isem.at[bi]),
            pltpu.make_async_copy(nrm_hbm.at[pl.ds(base, C)], nrm_v.at[bi],
                                  isem.at[bi]),
        )

    def in_copies(i, bi):
        return (
            pltpu.make_async_copy(p_hbm.at[src_v.at[bi]], pg_v.at[bi],
                                  gsem.at[bi]),
            pltpu.make_async_copy(a_hbm.at[pl.ds(base0 + i * C, C), :],
                                  a_v.at[bi], asem.at[bi]),
        )

    def scatter_copy(bi, bd):
        return pltpu.make_async_copy(pg_v.at[bi], acc_sh.at[dst_v.at[bd]],
                                     ssem.at[bi])

    for cp in idx_copies(0, 0, 0) + idx_copies(1, 1, 1):
        cp.start()
    for cp in idx_copies(0, 0, 0):
        cp.wait()
    for cp in in_copies(0, 0):
        cp.start()

    def outer_body(k2, carry):
        for b0 in range(DRING):
            i = k2 * DRING + b0
            b = b0 % NBUF

            j2 = i + 2
            bi2 = (b0 + 2) % NBUF
            bd2 = (b0 + 2) % DRING

            @pl.when(j2 < nchunks)
            def _():
                for cp in idx_copies(j2, bi2, bd2):
                    cp.start()

            j1 = i + 1
            bi1 = (b0 + 1) % NBUF
            bd1 = (b0 + 1) % DRING
            bd1_old = (bd1 + DRING - NBUF) % DRING

            @pl.when(jnp.logical_and(j1 < nchunks, j1 >= NBUF))
            def _():
                scatter_copy(bi1, bd1_old).wait()

            @pl.when(j1 < nchunks)
            def _():
                for cp in idx_copies(j1, bi1, bd1):
                    cp.wait()
                for cp in in_copies(j1, bi1):
                    cp.start()

            @pl.when(i < nchunks)
            def _():
                for cp in in_copies(i, b):
                    cp.wait()

                @plsc.parallel_loop(0, C, step=1, unroll=4)
                def _(e):
                    nv = nrm_v[b, pl.ds((e >> 4) << 4, 16)]
                    nrm = _lane_splat(nv, e & 15)
                    for f in range(D // 16):
                        sl = pl.ds(f * 16, 16)
                        m = (
                            jnp.maximum(pg_v[b, e, sl] + a_v[b, e, sl], 0.0)
                            * nrm
                        )
                        pg_v[b, e, sl] = m
                pltpu.async_copy(pg_v.at[b], acc_sh.at[dst_v.at[b0]],
                                 ssem.at[b], add=True)

        return carry

    nouter = (nchunks + DRING - 1) // DRING
    lax.fori_loop(0, nouter, outer_body, 0)

    for i in range(nchunks - NBUF, nchunks):
        scatter_copy(i % NBUF, i % DRING).wait()

    plsc.subcore_barrier()
    pltpu.sync_copy(acc_sh.at[rsl, :], agg_hbm.at[c, rsl, :])


@functools.partial(
    pl.kernel,
    out_type=jax.ShapeDtypeStruct((NC, NPAD, D), jnp.float32),
    mesh=plsc.VectorSubcoreMesh(core_axis_name="c", subcore_axis_name="s"),
    scratch_types=[
        pltpu.VMEM_SHARED((NPAD, D), jnp.float32),
        pltpu.VMEM((NBUF, C), jnp.int32),
        pltpu.VMEM((DRING, C), jnp.int32),
        pltpu.VMEM((NBUF, C), jnp.float32),
        pltpu.VMEM((NBUF, C, D), jnp.float32),
        pltpu.VMEM((NBUF, C, D), jnp.float32),
        pltpu.SemaphoreType.DMA((NBUF,)),
        pltpu.SemaphoreType.DMA((NBUF,)),
        pltpu.SemaphoreType.DMA((NBUF,)),
        pltpu.SemaphoreType.DMA((NBUF,)),
    ],
)
def _sc_gather_scatter(p_hbm, a_hbm, src_hbm, dst_hbm, nrm_hbm, zeros_hbm,
                       agg_hbm, acc_sh, src_v, dst_v, nrm_v, pg_v, a_v,
                       isem, gsem, asem, ssem):
    _sc_body(p_hbm, a_hbm, src_hbm, dst_hbm, nrm_hbm, zeros_hbm, agg_hbm,
             acc_sh, src_v, dst_v, nrm_v, pg_v, a_v,
             isem, gsem, asem, ssem)


def _pass(x_src, x_dst, x_0, edge_index, edge_attr, edge_norm, wm, bm, wu, bu,
          zeros):
    p = _tc_linear(x_src, wm[:D], bm, block_rows=2000)
    a = _tc_linear(edge_attr, wm[D:], jnp.zeros((D,), jnp.float32),
                   block_rows=2000)
    src = edge_index[0].astype(jnp.int32)
    dst = edge_index[1].astype(jnp.int32)
    agg2 = _sc_gather_scatter(p, a, src, dst, edge_norm, zeros)
    return _tc_update(x_dst, agg2, wu, bu, x_0, block_rows=2000)


def kernel(cons_embedding, vals_embedding, cons_embedding_0, vals_embedding_0,
           v2c_edge_index, c2v_edge_index, v2c_edge_attr, c2v_edge_attr,
           cons_batch, vals_batch, edge_norm, Wm_v2c, bm_v2c, Wu_v2c, bu_v2c,
           Wm_c2v, bm_c2v, Wu_c2v, bu_c2v):
    del cons_batch, vals_batch
    zeros = jnp.zeros((NPAD, D), jnp.float32)
    cons_new = _pass(vals_embedding, cons_embedding, cons_embedding_0,
                     v2c_edge_index, v2c_edge_attr, edge_norm,
                     Wm_v2c, bm_v2c, Wu_v2c, bu_v2c, zeros)
    vals_new = _pass(cons_new, vals_embedding, vals_embedding_0,
                     c2v_edge_index, c2v_edge_attr, edge_norm,
                     Wm_c2v, bm_c2v, Wu_c2v, bu_c2v, zeros)
    return (vals_new, cons_new)
